# baseline (speedup 1.0000x reference)
import functools

import jax
import jax.numpy as jnp
from jax import lax
from jax.experimental import pallas as pl
from jax.experimental.pallas import tpu as pltpu
from jax.experimental.pallas import tpu_sc as plsc

B = 1024
CTX = 20
D = 64
V = 100000

NC = 2
NS = 16
L = 16
NW = NC * NS
D_PER_W = D // NW

_sc_mesh = plsc.VectorSubcoreMesh(core_axis_name="c", subcore_axis_name="s")


@functools.partial(
    pl.kernel,
    mesh=_sc_mesh,
    out_type=jax.ShapeDtypeStruct((D, B), jnp.float32),
    scratch_types=[
        pltpu.VMEM((CTX, B), jnp.int32),
        pltpu.VMEM((V,), jnp.float32),
        pltpu.VMEM((B,), jnp.float32),
        pltpu.SemaphoreType.DMA,
    ],
    compiler_params=pltpu.CompilerParams(needs_layout_passes=False),
)
def _gather_sum_t(idxT_hbm, tableT_hbm, xT_hbm, idx_v, row_v, xrow_v, sem):
    wid = lax.axis_index("s") * NC + lax.axis_index("c")
    pltpu.sync_copy(idxT_hbm, idx_v)
    for dd in range(D_PER_W):
        d = wid * D_PER_W + dd
        pltpu.sync_copy(tableT_hbm.at[d], row_v)

        def body(b0, carry):
            acc = plsc.load_gather(row_v, [idx_v[0, pl.ds(b0, L)]])
            for j in range(1, CTX):
                acc = acc + plsc.load_gather(row_v, [idx_v[j, pl.ds(b0, L)]])
            xrow_v[pl.ds(b0, L)] = acc
            return carry

        lax.fori_loop(0, B // L, lambda i, c: body(i * L, c), 0, unroll=2)
        pltpu.sync_copy(xrow_v, xT_hbm.at[d])


VB = 4096


def _proj_body(wt_ref, x_ref, b_ref, o_ref):
    acc = lax.dot_general(
        wt_ref[...], x_ref[...], (((0,), (0,)), ((), ())),
        preferred_element_type=jnp.float32,
    )
    o_ref[...] = acc + b_ref[...].T


def kernel(inputs, emb_table, W, b):
    idxT = inputs.astype(jnp.int32).T
    tableT = emb_table.T
    xT = _gather_sum_t(idxT, tableT)

    nblk = (V + VB - 1) // VB
    outT = pl.pallas_call(
        _proj_body,
        grid=(nblk,),
        in_specs=[
            pl.BlockSpec((D, VB), lambda i: (0, i)),
            pl.BlockSpec((D, B), lambda i: (0, 0)),
            pl.BlockSpec((1, VB), lambda i: (0, i)),
        ],
        out_specs=pl.BlockSpec((VB, B), lambda i: (i, 0)),
        out_shape=jax.ShapeDtypeStruct((V, B), jnp.float32),
    )(W.T, xT, b.reshape(1, V))
    return outT.T


# speedup vs baseline: 1.0072x; 1.0072x over previous
import functools

import jax
import jax.numpy as jnp
from jax import lax
from jax.experimental import pallas as pl
from jax.experimental.pallas import tpu as pltpu
from jax.experimental.pallas import tpu_sc as plsc

B = 1024
CTX = 20
D = 64
V = 100000

NC = 2
NS = 16
L = 16
NW = NC * NS
D_PER_W = D // NW

_sc_mesh = plsc.VectorSubcoreMesh(core_axis_name="c", subcore_axis_name="s")


@functools.partial(
    pl.kernel,
    mesh=_sc_mesh,
    out_type=jax.ShapeDtypeStruct((D, B), jnp.float32),
    scratch_types=[
        pltpu.VMEM((CTX, B), jnp.int32),
        pltpu.VMEM((V,), jnp.float32),
        pltpu.VMEM((B,), jnp.float32),
        pltpu.SemaphoreType.DMA,
    ],
    compiler_params=pltpu.CompilerParams(needs_layout_passes=False),
)
def _gather_sum_t(idxT_hbm, tableT_hbm, xT_hbm, idx_v, row_v, xrow_v, sem):
    wid = lax.axis_index("s") * NC + lax.axis_index("c")
    pltpu.sync_copy(idxT_hbm, idx_v)
    for dd in range(D_PER_W):
        d = wid * D_PER_W + dd
        pltpu.sync_copy(tableT_hbm.at[d], row_v)

        def body(b0, carry):
            acc = plsc.load_gather(row_v, [idx_v[0, pl.ds(b0, L)]])
            for j in range(1, CTX):
                acc = acc + plsc.load_gather(row_v, [idx_v[j, pl.ds(b0, L)]])
            xrow_v[pl.ds(b0, L)] = acc
            return carry

        lax.fori_loop(0, B // L, lambda i, c: body(i * L, c), 0)
        pltpu.sync_copy(xrow_v, xT_hbm.at[d])


VB = 4096


def _proj_body(wt_ref, x_ref, b_ref, o_ref):
    acc = lax.dot_general(
        wt_ref[...], x_ref[...], (((0,), (0,)), ((), ())),
        preferred_element_type=jnp.float32,
    )
    o_ref[...] = acc + b_ref[...].T


def kernel(inputs, emb_table, W, b):
    idxT = inputs.astype(jnp.int32).T
    tableT = emb_table.T
    xT = _gather_sum_t(idxT, tableT)

    nblk = (V + VB - 1) // VB
    outT = pl.pallas_call(
        _proj_body,
        grid=(nblk,),
        in_specs=[
            pl.BlockSpec((D, VB), lambda i: (0, i)),
            pl.BlockSpec((D, B), lambda i: (0, 0)),
            pl.BlockSpec((1, VB), lambda i: (0, i)),
        ],
        out_specs=pl.BlockSpec((VB, B), lambda i: (i, 0)),
        out_shape=jax.ShapeDtypeStruct((V, B), jnp.float32),
    )(W.T, xT, b.reshape(1, V))
    return outT.T
